# Initial kernel scaffold; baseline (speedup 1.0000x reference)
import jax
import jax.numpy as jnp
from jax.experimental import pallas as pl


def kernel(u_embs, i_embs, edge_index, user_seq, W_ih0, W_hh0, b_ih0, b_hh0, W_ih1, W_hh1, b_ih1, b_hh1):
    raise NotImplementedError("write your pallas kernel here")


# speedup vs baseline: 1.6838x; 1.6838x over previous
import functools

import jax
import jax.numpy as jnp
from jax import lax
from jax.experimental import pallas as pl
from jax.experimental.pallas import tpu as pltpu

EMB = 128
N_NODES = 10000
N_PAD = 10016
E = 320000
E_PAD = 327680
B = 1024
L = 50
H3 = 3 * EMB


def _prep_body(degp_ref, u_ref, i_ref, a_ref, b_ref, x0_ref, xs0_ref):
    deg = jnp.sum(degp_ref[...], axis=0)
    ab = lax.rsqrt(jnp.maximum(deg, 1.0))
    a_col = jnp.reshape(ab[0, :], (N_PAD, 1))
    b_col = jnp.reshape(ab[1, :], (N_PAD, 1))
    a_ref[...] = a_col
    b_ref[...] = b_col
    zeros_pad = jnp.zeros((N_PAD - N_NODES, EMB), dtype=jnp.float32)
    x0 = jnp.concatenate([u_ref[...], i_ref[...], zeros_pad], axis=0)
    x0_ref[...] = x0
    xs0_ref[...] = a_col * x0


def _prep_call(deg_partials, u_embs, i_embs):
    return pl.pallas_call(
        _prep_body,
        out_shape=(
            jax.ShapeDtypeStruct((N_PAD, 1), jnp.float32),
            jax.ShapeDtypeStruct((N_PAD, 1), jnp.float32),
            jax.ShapeDtypeStruct((N_PAD, EMB), jnp.float32),
            jax.ShapeDtypeStruct((N_PAD, EMB), jnp.float32),
        ),
    )(deg_partials, u_embs, i_embs)


def _combine_body(p_ref, a_ref, b_ref, acc_ref, xs_ref, accout_ref):
    x = b_ref[...] * jnp.sum(p_ref[...], axis=0)
    xs_ref[...] = a_ref[...] * x
    accout_ref[...] = acc_ref[...] + x


def _combine_call(partials, a, b, acc):
    return pl.pallas_call(
        _combine_body,
        out_shape=(
            jax.ShapeDtypeStruct((N_PAD, EMB), jnp.float32),
            jax.ShapeDtypeStruct((N_PAD, EMB), jnp.float32),
        ),
    )(partials, a, b, acc)


def _final_body(p_ref, b_ref, acc_ref, emb_ref):
    x2 = b_ref[...] * jnp.sum(p_ref[...], axis=0)
    emb_ref[...] = (acc_ref[...] + x2) * (1.0 / 3.0)


def _final_call(partials, b, acc):
    return pl.pallas_call(
        _final_body,
        out_shape=jax.ShapeDtypeStruct((N_PAD, EMB), jnp.float32),
    )(partials, b, acc)


def _gru_body(x_ref, wih0_ref, whh0_ref, bih0_ref, bhh0_ref,
              wih1_ref, bih1_ref, bhh1_ref, out_ref, h_ref):
    t = pl.program_id(0)

    @pl.when(t == 0)
    def _():
        h_ref[...] = jnp.zeros_like(h_ref)

    h = h_ref[...]
    gi = jnp.dot(x_ref[...], wih0_ref[...],
                 preferred_element_type=jnp.float32) + bih0_ref[...]
    gh = jnp.dot(h, whh0_ref[...],
                 preferred_element_type=jnp.float32) + bhh0_ref[...]
    r = jax.nn.sigmoid(gi[:, 0:EMB] + gh[:, 0:EMB])
    z = jax.nn.sigmoid(gi[:, EMB:2 * EMB] + gh[:, EMB:2 * EMB])
    n = jnp.tanh(gi[:, 2 * EMB:] + r * gh[:, 2 * EMB:])
    h_new = (1.0 - z) * n + z * h
    h_ref[...] = h_new

    @pl.when(t == pl.num_programs(0) - 1)
    def _():
        gi1 = jnp.dot(h_new, wih1_ref[...],
                      preferred_element_type=jnp.float32) + bih1_ref[...]
        r1 = jax.nn.sigmoid(gi1[:, 0:EMB] + bhh1_ref[:, 0:EMB])
        z1 = jax.nn.sigmoid(gi1[:, EMB:2 * EMB] + bhh1_ref[:, EMB:2 * EMB])
        n1 = jnp.tanh(gi1[:, 2 * EMB:] + r1 * bhh1_ref[:, 2 * EMB:])
        out_ref[...] = (1.0 - z1) * n1


def _gru_call(de_emb_flat, wih0t, whh0t, bih0, bhh0, wih1t, bih1, bhh1):
    full = lambda s: pl.BlockSpec(s, lambda t: (0,) * len(s))
    return pl.pallas_call(
        _gru_body,
        grid=(L,),
        in_specs=[
            pl.BlockSpec((B, EMB), lambda t: (t, 0)),
            full((EMB, H3)), full((EMB, H3)), full((1, H3)), full((1, H3)),
            full((EMB, H3)), full((1, H3)), full((1, H3)),
        ],
        out_specs=pl.BlockSpec((B, EMB), lambda t: (0, 0)),
        out_shape=jax.ShapeDtypeStruct((B, EMB), jnp.float32),
        scratch_shapes=[pltpu.VMEM((B, EMB), jnp.float32)],
    )(de_emb_flat, wih0t, whh0t, bih0, bhh0, wih1t, bih1, bhh1)


def _degrees(src_p, dst_p):
    ones = jnp.ones((E_PAD,), dtype=jnp.float32)
    dego = jax.ops.segment_sum(ones, src_p, num_segments=N_PAD)
    degi = jax.ops.segment_sum(ones, dst_p, num_segments=N_PAD)
    return jnp.stack([dego, degi])[None]


def _smooth_partials(xs, src_p, dst_p):
    msgs = xs[src_p]
    return jax.ops.segment_sum(msgs, dst_p, num_segments=N_PAD)[None]


def _seq_gather(emb, seq_flat):
    return emb[seq_flat]


def kernel(u_embs, i_embs, edge_index, user_seq,
           W_ih0, W_hh0, b_ih0, b_hh0, W_ih1, W_hh1, b_ih1, b_hh1):
    del W_hh1
    edge32 = edge_index.astype(jnp.int32)
    pad = 10000 + (jnp.arange(E_PAD - E, dtype=jnp.int32) % 16)
    src_p = jnp.concatenate([edge32[0], pad])
    dst_p = jnp.concatenate([edge32[1], pad])

    deg_partials = _degrees(src_p, dst_p)
    a, b, x0, xs0 = _prep_call(deg_partials, u_embs, i_embs)

    p1 = _smooth_partials(xs0, src_p, dst_p)
    xs1, acc1 = _combine_call(p1, a, b, x0)
    p2 = _smooth_partials(xs1, src_p, dst_p)
    emb_pad = _final_call(p2, b, acc1)

    seq_flat = user_seq.astype(jnp.int32).T.reshape(-1)
    de_emb = _seq_gather(emb_pad, seq_flat)

    h = _gru_call(
        de_emb,
        W_ih0.T, W_hh0.T, b_ih0.reshape(1, H3), b_hh0.reshape(1, H3),
        W_ih1.T, b_ih1.reshape(1, H3), b_hh1.reshape(1, H3),
    )
    return (h, emb_pad[:N_NODES])


# speedup vs baseline: 9.0780x; 5.3914x over previous
import functools

import jax
import jax.numpy as jnp
from jax import lax
from jax.experimental import pallas as pl
from jax.experimental.pallas import tpu as pltpu
from jax.experimental.pallas import tpu_sc as plsc

EMB = 128
N_NODES = 10000
N_PAD = 10240
E = 320000
E_PAD = 327680
EROWS = E_PAD // 128
NC, NS = 2, 16
NW = NC * NS
RPW = EROWS // NW
TROWS = N_PAD // NS
B = 1024
L = 50
SEQROWS = (L * B) // 128
H3 = 3 * EMB

_mesh = plsc.VectorSubcoreMesh(core_axis_name="c", subcore_axis_name="s")


@functools.partial(
    pl.kernel,
    out_type=jax.ShapeDtypeStruct((NC, N_PAD, EMB), jnp.float32),
    mesh=_mesh,
    scratch_types=[
        pltpu.VMEM((128,), jnp.int32),
        pltpu.VMEM((128, EMB), jnp.float32),
        pltpu.VMEM_SHARED((N_PAD, EMB), jnp.float32),
    ],
)
def _degrees_sc(e_hbm, ones_hbm, zz2_hbm, out_hbm, idx_v, ones_v, acc_sh):
    c = lax.axis_index("c")
    s = lax.axis_index("s")
    pltpu.sync_copy(ones_hbm, ones_v)
    pltpu.sync_copy(zz2_hbm, acc_sh.at[pl.ds(s * TROWS, TROWS)])
    plsc.subcore_barrier()
    rpt = EROWS // NS

    def body(k, _):
        r = s * rpt + k
        pltpu.sync_copy(e_hbm.at[c, r], idx_v)
        pltpu.sync_copy(ones_v, acc_sh.at[idx_v], add=True)
        return _

    lax.fori_loop(0, rpt, body, 0)
    plsc.subcore_barrier()
    pltpu.sync_copy(acc_sh.at[pl.ds(s * TROWS, TROWS)],
                    out_hbm.at[c, pl.ds(s * TROWS, TROWS)])


@functools.partial(
    pl.kernel,
    out_type=jax.ShapeDtypeStruct((NC, N_PAD, EMB), jnp.float32),
    mesh=_mesh,
    scratch_types=[
        pltpu.VMEM((128,), jnp.int32),
        pltpu.VMEM((128,), jnp.int32),
        pltpu.VMEM((128, EMB), jnp.float32),
        pltpu.VMEM_SHARED((N_PAD, EMB), jnp.float32),
        pltpu.SemaphoreType.DMA,
    ],
)
def _smooth_sc(xs_hbm, src_hbm, dst_hbm, zz2_hbm, out_hbm,
               sidx, didx, rows, acc_sh, sem):
    c = lax.axis_index("c")
    s = lax.axis_index("s")
    wid = c * NS + s
    pltpu.sync_copy(zz2_hbm, acc_sh.at[pl.ds(s * TROWS, TROWS)])
    plsc.subcore_barrier()

    def body(k, _):
        r = wid * RPW + k
        pltpu.sync_copy(src_hbm.at[r], sidx)
        pltpu.sync_copy(dst_hbm.at[r], didx)
        pltpu.async_copy(xs_hbm.at[sidx], rows, sem).wait()
        pltpu.sync_copy(rows, acc_sh.at[didx], add=True)
        return _

    lax.fori_loop(0, RPW, body, 0)
    plsc.subcore_barrier()
    pltpu.sync_copy(acc_sh.at[pl.ds(s * TROWS, TROWS)],
                    out_hbm.at[c, pl.ds(s * TROWS, TROWS)])


@functools.partial(
    pl.kernel,
    out_type=jax.ShapeDtypeStruct((L * B, EMB), jnp.float32),
    mesh=_mesh,
    scratch_types=[
        pltpu.VMEM((128,), jnp.int32),
        pltpu.VMEM((128, EMB), jnp.float32),
        pltpu.SemaphoreType.DMA,
    ],
)
def _seq_gather_sc(emb_hbm, seq_hbm, out_hbm, idx_v, rows, sem):
    c = lax.axis_index("c")
    s = lax.axis_index("s")
    wid = c * NS + s
    nb = jnp.where(wid < 16, 13, 12)
    base = jnp.where(wid < 16, wid * 13, 208 + (wid - 16) * 12)

    def body(k, _):
        r = base + k
        pltpu.sync_copy(seq_hbm.at[r], idx_v)
        pltpu.async_copy(emb_hbm.at[idx_v], rows, sem).wait()
        pltpu.sync_copy(rows, out_hbm.at[pl.ds(r * 128, 128)])
        return _

    lax.fori_loop(0, nb, body, 0)


def _prep_body(degp_ref, u_ref, i_ref, a_ref, b_ref, x0_ref, xs0_ref):
    a_col = lax.rsqrt(jnp.maximum(degp_ref[0, :, 0:1], 1.0))
    b_col = lax.rsqrt(jnp.maximum(degp_ref[1, :, 0:1], 1.0))
    a_ref[...] = a_col
    b_ref[...] = b_col
    zeros_pad = jnp.zeros((N_PAD - N_NODES, EMB), dtype=jnp.float32)
    x0 = jnp.concatenate([u_ref[...], i_ref[...], zeros_pad], axis=0)
    x0_ref[...] = x0
    xs0_ref[...] = a_col * x0


def _prep_call(deg_partials, u_embs, i_embs):
    return pl.pallas_call(
        _prep_body,
        out_shape=(
            jax.ShapeDtypeStruct((N_PAD, 1), jnp.float32),
            jax.ShapeDtypeStruct((N_PAD, 1), jnp.float32),
            jax.ShapeDtypeStruct((N_PAD, EMB), jnp.float32),
            jax.ShapeDtypeStruct((N_PAD, EMB), jnp.float32),
        ),
    )(deg_partials, u_embs, i_embs)


def _combine_body(p_ref, a_ref, b_ref, acc_ref, xs_ref, accout_ref):
    x = b_ref[...] * jnp.sum(p_ref[...], axis=0)
    xs_ref[...] = a_ref[...] * x
    accout_ref[...] = acc_ref[...] + x


def _combine_call(partials, a, b, acc):
    return pl.pallas_call(
        _combine_body,
        out_shape=(
            jax.ShapeDtypeStruct((N_PAD, EMB), jnp.float32),
            jax.ShapeDtypeStruct((N_PAD, EMB), jnp.float32),
        ),
    )(partials, a, b, acc)


def _final_body(p_ref, b_ref, acc_ref, emb_ref):
    x2 = b_ref[...] * jnp.sum(p_ref[...], axis=0)
    emb_ref[...] = (acc_ref[...] + x2) * (1.0 / 3.0)


def _final_call(partials, b, acc):
    return pl.pallas_call(
        _final_body,
        out_shape=jax.ShapeDtypeStruct((N_PAD, EMB), jnp.float32),
    )(partials, b, acc)


def _gru_body(x_ref, wih0_ref, whh0_ref, bih0_ref, bhh0_ref,
              wih1_ref, bih1_ref, bhh1_ref, out_ref, h_ref):
    t = pl.program_id(0)

    @pl.when(t == 0)
    def _():
        h_ref[...] = jnp.zeros_like(h_ref)

    h = h_ref[...]
    gi = jnp.dot(x_ref[...], wih0_ref[...],
                 preferred_element_type=jnp.float32) + bih0_ref[...]
    gh = jnp.dot(h, whh0_ref[...],
                 preferred_element_type=jnp.float32) + bhh0_ref[...]
    r = jax.nn.sigmoid(gi[:, 0:EMB] + gh[:, 0:EMB])
    z = jax.nn.sigmoid(gi[:, EMB:2 * EMB] + gh[:, EMB:2 * EMB])
    n = jnp.tanh(gi[:, 2 * EMB:] + r * gh[:, 2 * EMB:])
    h_new = (1.0 - z) * n + z * h
    h_ref[...] = h_new

    @pl.when(t == pl.num_programs(0) - 1)
    def _():
        gi1 = jnp.dot(h_new, wih1_ref[...],
                      preferred_element_type=jnp.float32) + bih1_ref[...]
        r1 = jax.nn.sigmoid(gi1[:, 0:EMB] + bhh1_ref[:, 0:EMB])
        z1 = jax.nn.sigmoid(gi1[:, EMB:2 * EMB] + bhh1_ref[:, EMB:2 * EMB])
        n1 = jnp.tanh(gi1[:, 2 * EMB:] + r1 * bhh1_ref[:, 2 * EMB:])
        out_ref[...] = (1.0 - z1) * n1


def _gru_call(de_emb_flat, wih0t, whh0t, bih0, bhh0, wih1t, bih1, bhh1):
    full = lambda s: pl.BlockSpec(s, lambda t: (0,) * len(s))
    return pl.pallas_call(
        _gru_body,
        grid=(L,),
        in_specs=[
            pl.BlockSpec((B, EMB), lambda t: (t, 0)),
            full((EMB, H3)), full((EMB, H3)), full((1, H3)), full((1, H3)),
            full((EMB, H3)), full((1, H3)), full((1, H3)),
        ],
        out_specs=pl.BlockSpec((B, EMB), lambda t: (0, 0)),
        out_shape=jax.ShapeDtypeStruct((B, EMB), jnp.float32),
        scratch_shapes=[pltpu.VMEM((B, EMB), jnp.float32)],
    )(de_emb_flat, wih0t, whh0t, bih0, bhh0, wih1t, bih1, bhh1)


def kernel(u_embs, i_embs, edge_index, user_seq,
           W_ih0, W_hh0, b_ih0, b_hh0, W_ih1, W_hh1, b_ih1, b_hh1):
    del W_hh1
    edge32 = edge_index.astype(jnp.int32)
    pad = 10000 + (jnp.arange(E_PAD - E, dtype=jnp.int32) % (N_PAD - N_NODES))
    src2d = jnp.concatenate([edge32[0], pad]).reshape(EROWS, 128)
    dst2d = jnp.concatenate([edge32[1], pad]).reshape(EROWS, 128)
    e2d = jnp.stack([src2d, dst2d])
    zz = jnp.zeros((TROWS, EMB), jnp.float32)
    ones128 = jnp.ones((128, EMB), jnp.float32)

    deg_partials = _degrees_sc(e2d, ones128, zz)
    a, b, x0, xs0 = _prep_call(deg_partials, u_embs, i_embs)

    p1 = _smooth_sc(xs0, src2d, dst2d, zz)
    xs1, acc1 = _combine_call(p1, a, b, x0)
    p2 = _smooth_sc(xs1, src2d, dst2d, zz)
    emb_pad = _final_call(p2, b, acc1)

    seq2d = user_seq.astype(jnp.int32).T.reshape(SEQROWS, 128)
    de_emb = _seq_gather_sc(emb_pad, seq2d)

    h = _gru_call(
        de_emb,
        W_ih0.T, W_hh0.T, b_ih0.reshape(1, H3), b_hh0.reshape(1, H3),
        W_ih1.T, b_ih1.reshape(1, H3), b_hh1.reshape(1, H3),
    )
    return (h, emb_pad[:N_NODES])


# speedup vs baseline: 12.4749x; 1.3742x over previous
import functools

import jax
import jax.numpy as jnp
from jax import lax
from jax.experimental import pallas as pl
from jax.experimental.pallas import tpu as pltpu
from jax.experimental.pallas import tpu_sc as plsc

EMB = 128
N_NODES = 10000
N_PAD = 10240
E = 320000
E_PAD = 327680
EROWS = E_PAD // 128
NC, NS = 2, 16
NW = NC * NS
RPW = EROWS // NW
TROWS = N_PAD // NS
B = 1024
L = 50
SEQROWS = (L * B) // 128
H3 = 3 * EMB

_mesh = plsc.VectorSubcoreMesh(core_axis_name="c", subcore_axis_name="s")


@functools.partial(
    pl.kernel,
    out_type=jax.ShapeDtypeStruct((NC, N_PAD, EMB), jnp.float32),
    mesh=_mesh,
    scratch_types=[
        pltpu.VMEM((128,), jnp.int32),
        pltpu.VMEM((128, EMB), jnp.float32),
        pltpu.VMEM_SHARED((N_PAD, EMB), jnp.float32),
    ],
)
def _degrees_sc(e_hbm, ones_hbm, zz2_hbm, out_hbm, idx_v, ones_v, acc_sh):
    c = lax.axis_index("c")
    s = lax.axis_index("s")
    pltpu.sync_copy(ones_hbm, ones_v)
    pltpu.sync_copy(zz2_hbm, acc_sh.at[pl.ds(s * TROWS, TROWS)])
    plsc.subcore_barrier()
    rpt = EROWS // NS

    def body(k, _):
        r = s * rpt + k
        pltpu.sync_copy(e_hbm.at[c, r], idx_v)
        pltpu.sync_copy(ones_v, acc_sh.at[idx_v], add=True)
        return _

    lax.fori_loop(0, rpt, body, 0)
    plsc.subcore_barrier()
    pltpu.sync_copy(acc_sh.at[pl.ds(s * TROWS, TROWS)],
                    out_hbm.at[c, pl.ds(s * TROWS, TROWS)])


@functools.partial(
    pl.kernel,
    out_type=jax.ShapeDtypeStruct((NC, N_PAD, EMB), jnp.float32),
    mesh=_mesh,
    scratch_types=[
        pltpu.VMEM((4, 128), jnp.int32),
        pltpu.VMEM((4, 128), jnp.int32),
        pltpu.VMEM((2, 128, EMB), jnp.float32),
        pltpu.VMEM_SHARED((N_PAD, EMB), jnp.float32),
        pltpu.SemaphoreType.DMA,
        pltpu.SemaphoreType.DMA,
        pltpu.SemaphoreType.DMA,
        pltpu.SemaphoreType.DMA,
    ],
)
def _smooth_sc(xs_hbm, src_hbm, dst_hbm, zz2_hbm, out_hbm,
               idx4, didx4, rows2, acc_sh, sem_i, sem_g, sem_s0, sem_s1):
    c = lax.axis_index("c")
    s = lax.axis_index("s")
    wid = c * NS + s
    base = wid * RPW
    pltpu.sync_copy(zz2_hbm, acc_sh.at[pl.ds(s * TROWS, TROWS)])
    plsc.subcore_barrier()

    pltpu.async_copy(src_hbm.at[base], idx4.at[0], sem_i)
    pltpu.async_copy(dst_hbm.at[base], didx4.at[0], sem_i)

    def chunk(k, q, first, pf, drain):
        b = q % 2
        sem_s = sem_s0 if b == 0 else sem_s1
        if not first:
            pltpu.make_async_copy(rows2.at[b], acc_sh.at[didx4.at[(q + 2) % 4]],
                                  sem_s).wait()
        pltpu.make_async_copy(src_hbm.at[base], idx4.at[q], sem_i).wait()
        pltpu.make_async_copy(dst_hbm.at[base], didx4.at[q], sem_i).wait()

        if pf:
            pltpu.async_copy(src_hbm.at[base + k + 1], idx4.at[(q + 1) % 4],
                             sem_i)
            pltpu.async_copy(dst_hbm.at[base + k + 1], didx4.at[(q + 1) % 4],
                             sem_i)

        pltpu.async_copy(xs_hbm.at[idx4.at[q]], rows2.at[b], sem_g).wait()
        cp = pltpu.async_copy(rows2.at[b], acc_sh.at[didx4.at[q]], sem_s,
                              add=True)
        if drain:
            cp.wait()

    def body(o, _):
        k = 2 + 4 * o
        chunk(k, 2, first=False, pf=True, drain=False)
        chunk(k + 1, 3, first=False, pf=True, drain=False)
        chunk(k + 2, 0, first=False, pf=True, drain=False)
        chunk(k + 3, 1, first=False, pf=True, drain=False)
        return _

    chunk(0, 0, first=True, pf=True, drain=False)
    chunk(1, 1, first=True, pf=True, drain=False)
    lax.fori_loop(0, (RPW - 4) // 4, body, 0)
    chunk(RPW - 2, 2, first=False, pf=True, drain=False)
    chunk(RPW - 1, 3, first=False, pf=False, drain=False)
    pltpu.make_async_copy(rows2.at[0], acc_sh.at[didx4.at[2]], sem_s0).wait()
    pltpu.make_async_copy(rows2.at[1], acc_sh.at[didx4.at[3]], sem_s1).wait()
    plsc.subcore_barrier()
    pltpu.sync_copy(acc_sh.at[pl.ds(s * TROWS, TROWS)],
                    out_hbm.at[c, pl.ds(s * TROWS, TROWS)])


@functools.partial(
    pl.kernel,
    out_type=jax.ShapeDtypeStruct((L * B, EMB), jnp.float32),
    mesh=_mesh,
    scratch_types=[
        pltpu.VMEM((128,), jnp.int32),
        pltpu.VMEM((128, EMB), jnp.float32),
        pltpu.SemaphoreType.DMA,
    ],
)
def _seq_gather_sc(emb_hbm, seq_hbm, out_hbm, idx_v, rows, sem):
    c = lax.axis_index("c")
    s = lax.axis_index("s")
    wid = c * NS + s
    nb = jnp.where(wid < 16, 13, 12)
    base = jnp.where(wid < 16, wid * 13, 208 + (wid - 16) * 12)

    def body(k, _):
        r = base + k
        pltpu.sync_copy(seq_hbm.at[r], idx_v)
        pltpu.async_copy(emb_hbm.at[idx_v], rows, sem).wait()
        pltpu.sync_copy(rows, out_hbm.at[pl.ds(r * 128, 128)])
        return _

    lax.fori_loop(0, nb, body, 0)


def _prep_body(degp_ref, u_ref, i_ref, a_ref, b_ref, x0_ref, xs0_ref):
    a_col = lax.rsqrt(jnp.maximum(degp_ref[0, :, 0:1], 1.0))
    b_col = lax.rsqrt(jnp.maximum(degp_ref[1, :, 0:1], 1.0))
    a_ref[...] = a_col
    b_ref[...] = b_col
    zeros_pad = jnp.zeros((N_PAD - N_NODES, EMB), dtype=jnp.float32)
    x0 = jnp.concatenate([u_ref[...], i_ref[...], zeros_pad], axis=0)
    x0_ref[...] = x0
    xs0_ref[...] = a_col * x0


def _prep_call(deg_partials, u_embs, i_embs):
    return pl.pallas_call(
        _prep_body,
        out_shape=(
            jax.ShapeDtypeStruct((N_PAD, 1), jnp.float32),
            jax.ShapeDtypeStruct((N_PAD, 1), jnp.float32),
            jax.ShapeDtypeStruct((N_PAD, EMB), jnp.float32),
            jax.ShapeDtypeStruct((N_PAD, EMB), jnp.float32),
        ),
    )(deg_partials, u_embs, i_embs)


def _combine_body(p_ref, a_ref, b_ref, acc_ref, xs_ref, accout_ref):
    x = b_ref[...] * jnp.sum(p_ref[...], axis=0)
    xs_ref[...] = a_ref[...] * x
    accout_ref[...] = acc_ref[...] + x


def _combine_call(partials, a, b, acc):
    return pl.pallas_call(
        _combine_body,
        out_shape=(
            jax.ShapeDtypeStruct((N_PAD, EMB), jnp.float32),
            jax.ShapeDtypeStruct((N_PAD, EMB), jnp.float32),
        ),
    )(partials, a, b, acc)


def _final_body(p_ref, b_ref, acc_ref, emb_ref):
    x2 = b_ref[...] * jnp.sum(p_ref[...], axis=0)
    emb_ref[...] = (acc_ref[...] + x2) * (1.0 / 3.0)


def _final_call(partials, b, acc):
    return pl.pallas_call(
        _final_body,
        out_shape=jax.ShapeDtypeStruct((N_PAD, EMB), jnp.float32),
    )(partials, b, acc)


def _gru_body(x_ref, wih0_ref, whh0_ref, bih0_ref, bhh0_ref,
              wih1_ref, bih1_ref, bhh1_ref, out_ref, h_ref):
    t = pl.program_id(0)

    @pl.when(t == 0)
    def _():
        h_ref[...] = jnp.zeros_like(h_ref)

    h = h_ref[...]
    gi = jnp.dot(x_ref[...], wih0_ref[...],
                 preferred_element_type=jnp.float32) + bih0_ref[...]
    gh = jnp.dot(h, whh0_ref[...],
                 preferred_element_type=jnp.float32) + bhh0_ref[...]
    r = jax.nn.sigmoid(gi[:, 0:EMB] + gh[:, 0:EMB])
    z = jax.nn.sigmoid(gi[:, EMB:2 * EMB] + gh[:, EMB:2 * EMB])
    n = jnp.tanh(gi[:, 2 * EMB:] + r * gh[:, 2 * EMB:])
    h_new = (1.0 - z) * n + z * h
    h_ref[...] = h_new

    @pl.when(t == pl.num_programs(0) - 1)
    def _():
        gi1 = jnp.dot(h_new, wih1_ref[...],
                      preferred_element_type=jnp.float32) + bih1_ref[...]
        r1 = jax.nn.sigmoid(gi1[:, 0:EMB] + bhh1_ref[:, 0:EMB])
        z1 = jax.nn.sigmoid(gi1[:, EMB:2 * EMB] + bhh1_ref[:, EMB:2 * EMB])
        n1 = jnp.tanh(gi1[:, 2 * EMB:] + r1 * bhh1_ref[:, 2 * EMB:])
        out_ref[...] = (1.0 - z1) * n1


def _gru_call(de_emb_flat, wih0t, whh0t, bih0, bhh0, wih1t, bih1, bhh1):
    full = lambda s: pl.BlockSpec(s, lambda t: (0,) * len(s))
    return pl.pallas_call(
        _gru_body,
        grid=(L,),
        in_specs=[
            pl.BlockSpec((B, EMB), lambda t: (t, 0)),
            full((EMB, H3)), full((EMB, H3)), full((1, H3)), full((1, H3)),
            full((EMB, H3)), full((1, H3)), full((1, H3)),
        ],
        out_specs=pl.BlockSpec((B, EMB), lambda t: (0, 0)),
        out_shape=jax.ShapeDtypeStruct((B, EMB), jnp.float32),
        scratch_shapes=[pltpu.VMEM((B, EMB), jnp.float32)],
    )(de_emb_flat, wih0t, whh0t, bih0, bhh0, wih1t, bih1, bhh1)


def kernel(u_embs, i_embs, edge_index, user_seq,
           W_ih0, W_hh0, b_ih0, b_hh0, W_ih1, W_hh1, b_ih1, b_hh1):
    del W_hh1
    edge32 = edge_index.astype(jnp.int32)
    pad = 10000 + (jnp.arange(E_PAD - E, dtype=jnp.int32) % (N_PAD - N_NODES))
    src2d = jnp.concatenate([edge32[0], pad]).reshape(EROWS, 128)
    dst2d = jnp.concatenate([edge32[1], pad]).reshape(EROWS, 128)
    e2d = jnp.stack([src2d, dst2d])
    zz = jnp.zeros((TROWS, EMB), jnp.float32)
    ones128 = jnp.ones((128, EMB), jnp.float32)

    deg_partials = _degrees_sc(e2d, ones128, zz)
    a, b, x0, xs0 = _prep_call(deg_partials, u_embs, i_embs)

    p1 = _smooth_sc(xs0, src2d, dst2d, zz)
    xs1, acc1 = _combine_call(p1, a, b, x0)
    p2 = _smooth_sc(xs1, src2d, dst2d, zz)
    emb_pad = _final_call(p2, b, acc1)

    seq2d = user_seq.astype(jnp.int32).T.reshape(SEQROWS, 128)
    de_emb = _seq_gather_sc(emb_pad, seq2d)

    h = _gru_call(
        de_emb,
        W_ih0.T, W_hh0.T, b_ih0.reshape(1, H3), b_hh0.reshape(1, H3),
        W_ih1.T, b_ih1.reshape(1, H3), b_hh1.reshape(1, H3),
    )
    return (h, emb_pad[:N_NODES])


# speedup vs baseline: 15.3981x; 1.2343x over previous
import functools

import jax
import jax.numpy as jnp
from jax import lax
from jax.experimental import pallas as pl
from jax.experimental.pallas import tpu as pltpu
from jax.experimental.pallas import tpu_sc as plsc

EMB = 128
N_NODES = 10000
N_PAD = 10240
E = 320000
E_PAD = 327680
EROWS = E_PAD // 128
NC, NS = 2, 16
NW = NC * NS
RPW = EROWS // NW
TROWS = N_PAD // NS
B = 1024
L = 50
SEQROWS = (L * B) // 128
H3 = 3 * EMB

_mesh = plsc.VectorSubcoreMesh(core_axis_name="c", subcore_axis_name="s")


HBINS = N_PAD // 2


@functools.partial(
    pl.kernel,
    out_type=jax.ShapeDtypeStruct((NW, 2, N_PAD), jnp.float32),
    mesh=_mesh,
    scratch_types=[
        pltpu.VMEM((RPW, 128), jnp.int32),
        pltpu.VMEM((16 * HBINS + 16,), jnp.float32),
        pltpu.VMEM((HBINS,), jnp.float32),
    ],
    compiler_params=pltpu.CompilerParams(needs_layout_passes=False),
)
def _degrees_sc(src_hbm, dst_hbm, zeros_hbm, out_hbm, ibig, hist, outbuf):
    c = lax.axis_index("c")
    s = lax.axis_index("s")
    wid = c * NS + s
    base = wid * RPW
    ones16 = jnp.ones((16,), jnp.float32)
    lane_ids = lax.iota(jnp.int32, 16)
    lane_base = lane_ids * HBINS

    for d, eref in ((0, src_hbm), (1, dst_hbm)):
        pltpu.sync_copy(eref.at[pl.ds(base, RPW)], ibig)
        for half in range(2):
            pltpu.sync_copy(zeros_hbm, hist)

            def scan(k, _):
                for j in range(8):
                    v = ibig[k, pl.ds(j * 16, 16)]
                    local = v - (half * HBINS)
                    m = (local >= 0) & (local < HBINS)
                    addr = jnp.where(m, local + lane_base,
                                     16 * HBINS + lane_ids)
                    plsc.addupdate_scatter(hist, [addr], ones16)
                return _

            lax.fori_loop(0, RPW, scan, 0)

            def reduce(g, _):
                acc = hist[pl.ds(g * 16, 16)]
                for j in range(1, 16):
                    acc = acc + hist[pl.ds(j * HBINS + g * 16, 16)]
                outbuf[pl.ds(g * 16, 16)] = acc
                return _

            lax.fori_loop(0, HBINS // 16, reduce, 0)
            pltpu.sync_copy(outbuf,
                            out_hbm.at[wid, d, pl.ds(half * HBINS, HBINS)])


@functools.partial(
    pl.kernel,
    out_type=jax.ShapeDtypeStruct((NC, N_PAD, EMB), jnp.float32),
    mesh=_mesh,
    scratch_types=[
        pltpu.VMEM((4, 128), jnp.int32),
        pltpu.VMEM((4, 128), jnp.int32),
        pltpu.VMEM((2, 128, EMB), jnp.float32),
        pltpu.VMEM_SHARED((N_PAD, EMB), jnp.float32),
        pltpu.SemaphoreType.DMA,
        pltpu.SemaphoreType.DMA,
        pltpu.SemaphoreType.DMA,
        pltpu.SemaphoreType.DMA,
    ],
)
def _smooth_sc(xs_hbm, src_hbm, dst_hbm, zz2_hbm, out_hbm,
               idx4, didx4, rows2, acc_sh, sem_i, sem_g, sem_s0, sem_s1):
    c = lax.axis_index("c")
    s = lax.axis_index("s")
    wid = c * NS + s
    base = wid * RPW
    pltpu.sync_copy(zz2_hbm, acc_sh.at[pl.ds(s * TROWS, TROWS)])
    plsc.subcore_barrier()

    pltpu.async_copy(src_hbm.at[base], idx4.at[0], sem_i)
    pltpu.async_copy(dst_hbm.at[base], didx4.at[0], sem_i)

    def chunk(k, q, first, pf, drain):
        b = q % 2
        sem_s = sem_s0 if b == 0 else sem_s1
        if not first:
            pltpu.make_async_copy(rows2.at[b], acc_sh.at[didx4.at[(q + 2) % 4]],
                                  sem_s).wait()
        pltpu.make_async_copy(src_hbm.at[base], idx4.at[q], sem_i).wait()
        pltpu.make_async_copy(dst_hbm.at[base], didx4.at[q], sem_i).wait()

        if pf:
            pltpu.async_copy(src_hbm.at[base + k + 1], idx4.at[(q + 1) % 4],
                             sem_i)
            pltpu.async_copy(dst_hbm.at[base + k + 1], didx4.at[(q + 1) % 4],
                             sem_i)

        pltpu.async_copy(xs_hbm.at[idx4.at[q]], rows2.at[b], sem_g).wait()
        cp = pltpu.async_copy(rows2.at[b], acc_sh.at[didx4.at[q]], sem_s,
                              add=True)
        if drain:
            cp.wait()

    def body(o, _):
        k = 2 + 4 * o
        chunk(k, 2, first=False, pf=True, drain=False)
        chunk(k + 1, 3, first=False, pf=True, drain=False)
        chunk(k + 2, 0, first=False, pf=True, drain=False)
        chunk(k + 3, 1, first=False, pf=True, drain=False)
        return _

    chunk(0, 0, first=True, pf=True, drain=False)
    chunk(1, 1, first=True, pf=True, drain=False)
    lax.fori_loop(0, (RPW - 4) // 4, body, 0)
    chunk(RPW - 2, 2, first=False, pf=True, drain=False)
    chunk(RPW - 1, 3, first=False, pf=False, drain=False)
    pltpu.make_async_copy(rows2.at[0], acc_sh.at[didx4.at[2]], sem_s0).wait()
    pltpu.make_async_copy(rows2.at[1], acc_sh.at[didx4.at[3]], sem_s1).wait()
    plsc.subcore_barrier()
    pltpu.sync_copy(acc_sh.at[pl.ds(s * TROWS, TROWS)],
                    out_hbm.at[c, pl.ds(s * TROWS, TROWS)])


@functools.partial(
    pl.kernel,
    out_type=jax.ShapeDtypeStruct((L * B, EMB), jnp.float32),
    mesh=_mesh,
    scratch_types=[
        pltpu.VMEM((128,), jnp.int32),
        pltpu.VMEM((128, EMB), jnp.float32),
        pltpu.SemaphoreType.DMA,
    ],
)
def _seq_gather_sc(emb_hbm, seq_hbm, out_hbm, idx_v, rows, sem):
    c = lax.axis_index("c")
    s = lax.axis_index("s")
    wid = c * NS + s
    nb = jnp.where(wid < 16, 13, 12)
    base = jnp.where(wid < 16, wid * 13, 208 + (wid - 16) * 12)

    def body(k, _):
        r = base + k
        pltpu.sync_copy(seq_hbm.at[r], idx_v)
        pltpu.async_copy(emb_hbm.at[idx_v], rows, sem).wait()
        pltpu.sync_copy(rows, out_hbm.at[pl.ds(r * 128, 128)])
        return _

    lax.fori_loop(0, nb, body, 0)


def _prep_body(degp_ref, u_ref, i_ref, a_ref, b_ref, x0_ref, xs0_ref):
    deg = jnp.sum(degp_ref[...], axis=0)
    ab = lax.rsqrt(jnp.maximum(deg, 1.0))
    a_col = jnp.reshape(ab[0, :], (N_PAD, 1))
    b_col = jnp.reshape(ab[1, :], (N_PAD, 1))
    a_ref[...] = a_col
    b_ref[...] = b_col
    zeros_pad = jnp.zeros((N_PAD - N_NODES, EMB), dtype=jnp.float32)
    x0 = jnp.concatenate([u_ref[...], i_ref[...], zeros_pad], axis=0)
    x0_ref[...] = x0
    xs0_ref[...] = a_col * x0


def _prep_call(deg_partials, u_embs, i_embs):
    return pl.pallas_call(
        _prep_body,
        out_shape=(
            jax.ShapeDtypeStruct((N_PAD, 1), jnp.float32),
            jax.ShapeDtypeStruct((N_PAD, 1), jnp.float32),
            jax.ShapeDtypeStruct((N_PAD, EMB), jnp.float32),
            jax.ShapeDtypeStruct((N_PAD, EMB), jnp.float32),
        ),
    )(deg_partials, u_embs, i_embs)


def _combine_body(p_ref, a_ref, b_ref, acc_ref, xs_ref, accout_ref):
    x = b_ref[...] * jnp.sum(p_ref[...], axis=0)
    xs_ref[...] = a_ref[...] * x
    accout_ref[...] = acc_ref[...] + x


def _combine_call(partials, a, b, acc):
    return pl.pallas_call(
        _combine_body,
        out_shape=(
            jax.ShapeDtypeStruct((N_PAD, EMB), jnp.float32),
            jax.ShapeDtypeStruct((N_PAD, EMB), jnp.float32),
        ),
    )(partials, a, b, acc)


def _final_body(p_ref, b_ref, acc_ref, emb_ref):
    x2 = b_ref[...] * jnp.sum(p_ref[...], axis=0)
    emb_ref[...] = (acc_ref[...] + x2) * (1.0 / 3.0)


def _final_call(partials, b, acc):
    return pl.pallas_call(
        _final_body,
        out_shape=jax.ShapeDtypeStruct((N_PAD, EMB), jnp.float32),
    )(partials, b, acc)


def _gru_body(x_ref, wih0_ref, whh0_ref, bih0_ref, bhh0_ref,
              wih1_ref, bih1_ref, bhh1_ref, out_ref, h_ref):
    t = pl.program_id(0)

    @pl.when(t == 0)
    def _():
        h_ref[...] = jnp.zeros_like(h_ref)

    h = h_ref[...]
    gi = jnp.dot(x_ref[...], wih0_ref[...],
                 preferred_element_type=jnp.float32) + bih0_ref[...]
    gh = jnp.dot(h, whh0_ref[...],
                 preferred_element_type=jnp.float32) + bhh0_ref[...]
    r = jax.nn.sigmoid(gi[:, 0:EMB] + gh[:, 0:EMB])
    z = jax.nn.sigmoid(gi[:, EMB:2 * EMB] + gh[:, EMB:2 * EMB])
    n = jnp.tanh(gi[:, 2 * EMB:] + r * gh[:, 2 * EMB:])
    h_new = (1.0 - z) * n + z * h
    h_ref[...] = h_new

    @pl.when(t == pl.num_programs(0) - 1)
    def _():
        gi1 = jnp.dot(h_new, wih1_ref[...],
                      preferred_element_type=jnp.float32) + bih1_ref[...]
        r1 = jax.nn.sigmoid(gi1[:, 0:EMB] + bhh1_ref[:, 0:EMB])
        z1 = jax.nn.sigmoid(gi1[:, EMB:2 * EMB] + bhh1_ref[:, EMB:2 * EMB])
        n1 = jnp.tanh(gi1[:, 2 * EMB:] + r1 * bhh1_ref[:, 2 * EMB:])
        out_ref[...] = (1.0 - z1) * n1


def _gru_call(de_emb_flat, wih0t, whh0t, bih0, bhh0, wih1t, bih1, bhh1):
    full = lambda s: pl.BlockSpec(s, lambda t: (0,) * len(s))
    return pl.pallas_call(
        _gru_body,
        grid=(L,),
        in_specs=[
            pl.BlockSpec((B, EMB), lambda t: (t, 0)),
            full((EMB, H3)), full((EMB, H3)), full((1, H3)), full((1, H3)),
            full((EMB, H3)), full((1, H3)), full((1, H3)),
        ],
        out_specs=pl.BlockSpec((B, EMB), lambda t: (0, 0)),
        out_shape=jax.ShapeDtypeStruct((B, EMB), jnp.float32),
        scratch_shapes=[pltpu.VMEM((B, EMB), jnp.float32)],
    )(de_emb_flat, wih0t, whh0t, bih0, bhh0, wih1t, bih1, bhh1)


def kernel(u_embs, i_embs, edge_index, user_seq,
           W_ih0, W_hh0, b_ih0, b_hh0, W_ih1, W_hh1, b_ih1, b_hh1):
    del W_hh1
    edge32 = edge_index.astype(jnp.int32)
    pad = 10000 + (jnp.arange(E_PAD - E, dtype=jnp.int32) % (N_PAD - N_NODES))
    src2d = jnp.concatenate([edge32[0], pad]).reshape(EROWS, 128)
    dst2d = jnp.concatenate([edge32[1], pad]).reshape(EROWS, 128)
    zz = jnp.zeros((TROWS, EMB), jnp.float32)
    zhist = jnp.zeros((16 * HBINS + 16,), jnp.float32)

    deg_partials = _degrees_sc(src2d, dst2d, zhist)
    a, b, x0, xs0 = _prep_call(deg_partials, u_embs, i_embs)

    p1 = _smooth_sc(xs0, src2d, dst2d, zz)
    xs1, acc1 = _combine_call(p1, a, b, x0)
    p2 = _smooth_sc(xs1, src2d, dst2d, zz)
    emb_pad = _final_call(p2, b, acc1)

    seq2d = user_seq.astype(jnp.int32).T.reshape(SEQROWS, 128)
    de_emb = _seq_gather_sc(emb_pad, seq2d)

    h = _gru_call(
        de_emb,
        W_ih0.T, W_hh0.T, b_ih0.reshape(1, H3), b_hh0.reshape(1, H3),
        W_ih1.T, b_ih1.reshape(1, H3), b_hh1.reshape(1, H3),
    )
    return (h, emb_pad[:N_NODES])


# speedup vs baseline: 17.0700x; 1.1086x over previous
import functools

import jax
import jax.numpy as jnp
from jax import lax
from jax.experimental import pallas as pl
from jax.experimental.pallas import tpu as pltpu
from jax.experimental.pallas import tpu_sc as plsc

EMB = 128
N_NODES = 10000
N_PAD = 10240
E = 320000
E_PAD = 327680
EROWS = E_PAD // 128
NC, NS = 2, 16
NW = NC * NS
RPW = EROWS // NW
TROWS = N_PAD // NS
B = 1024
L = 50
SEQROWS = (L * B) // 128
H3 = 3 * EMB

_mesh = plsc.VectorSubcoreMesh(core_axis_name="c", subcore_axis_name="s")


HBINS = N_PAD // 2


@functools.partial(
    pl.kernel,
    out_type=jax.ShapeDtypeStruct((NW, 2, N_PAD), jnp.float32),
    mesh=_mesh,
    scratch_types=[
        pltpu.VMEM((RPW, 128), jnp.int32),
        pltpu.VMEM((16 * HBINS + 16,), jnp.float32),
        pltpu.VMEM((HBINS,), jnp.float32),
    ],
    compiler_params=pltpu.CompilerParams(needs_layout_passes=False),
)
def _degrees_sc(src_hbm, dst_hbm, zeros_hbm, out_hbm, ibig, hist, outbuf):
    c = lax.axis_index("c")
    s = lax.axis_index("s")
    wid = c * NS + s
    base = wid * RPW
    ones16 = jnp.ones((16,), jnp.float32)
    lane_ids = lax.iota(jnp.int32, 16)
    lane_base = lane_ids * HBINS

    for d, eref in ((0, src_hbm), (1, dst_hbm)):
        pltpu.sync_copy(eref.at[pl.ds(base, RPW)], ibig)
        for half in range(2):
            pltpu.sync_copy(zeros_hbm, hist)

            def scan(k, _):
                for j in range(8):
                    v = ibig[k, pl.ds(j * 16, 16)]
                    local = v - (half * HBINS)
                    m = (local >= 0) & (local < HBINS)
                    addr = jnp.where(m, local + lane_base,
                                     16 * HBINS + lane_ids)
                    plsc.addupdate_scatter(hist, [addr], ones16)
                return _

            lax.fori_loop(0, RPW, scan, 0)

            def reduce(g, _):
                acc = hist[pl.ds(g * 16, 16)]
                for j in range(1, 16):
                    acc = acc + hist[pl.ds(j * HBINS + g * 16, 16)]
                outbuf[pl.ds(g * 16, 16)] = acc
                return _

            lax.fori_loop(0, HBINS // 16, reduce, 0)
            pltpu.sync_copy(outbuf,
                            out_hbm.at[wid, d, pl.ds(half * HBINS, HBINS)])


@functools.partial(
    pl.kernel,
    out_type=jax.ShapeDtypeStruct((NC, N_PAD, EMB), jnp.float32),
    mesh=_mesh,
    scratch_types=[
        pltpu.VMEM((4, 128), jnp.int32),
        pltpu.VMEM((4, 128), jnp.int32),
        pltpu.VMEM((2, 128, EMB), jnp.float32),
        pltpu.VMEM_SHARED((N_PAD, EMB), jnp.float32),
        pltpu.SemaphoreType.DMA,
        pltpu.SemaphoreType.DMA,
        pltpu.SemaphoreType.DMA,
        pltpu.SemaphoreType.DMA,
        pltpu.SemaphoreType.DMA,
    ],
)
def _smooth_sc(xs_hbm, src_hbm, dst_hbm, zz2_hbm, out_hbm,
               idx4, didx4, rows2, acc_sh, sem_i, sem_g0, sem_g1,
               sem_s0, sem_s1):
    c = lax.axis_index("c")
    s = lax.axis_index("s")
    wid = c * NS + s
    base = wid * RPW
    pltpu.sync_copy(zz2_hbm, acc_sh.at[pl.ds(s * TROWS, TROWS)])
    plsc.subcore_barrier()

    pltpu.async_copy(src_hbm.at[base], idx4.at[0], sem_i)
    pltpu.async_copy(dst_hbm.at[base], didx4.at[0], sem_i)

    sem_g = (sem_g0, sem_g1)
    sem_s = (sem_s0, sem_s1)

    def chunk(k, q, w_scat, pf, w_gath):
        b = q % 2
        qp = (q + 3) % 4
        if w_scat:
            pltpu.make_async_copy(rows2.at[b], acc_sh.at[didx4.at[(q + 2) % 4]],
                                  sem_s[b]).wait()
        pltpu.make_async_copy(src_hbm.at[base], idx4.at[q], sem_i).wait()
        pltpu.make_async_copy(dst_hbm.at[base], didx4.at[q], sem_i).wait()
        if pf:
            pltpu.async_copy(src_hbm.at[base + k + 1], idx4.at[(q + 1) % 4],
                             sem_i)
            pltpu.async_copy(dst_hbm.at[base + k + 1], didx4.at[(q + 1) % 4],
                             sem_i)
        pltpu.async_copy(xs_hbm.at[idx4.at[q]], rows2.at[b], sem_g[b])
        if w_gath:
            pltpu.make_async_copy(xs_hbm.at[idx4.at[qp]], rows2.at[1 - b],
                                  sem_g[1 - b]).wait()
            pltpu.async_copy(rows2.at[1 - b], acc_sh.at[didx4.at[qp]],
                             sem_s[1 - b], add=True)

    def body(o, _):
        k = 2 + 4 * o
        chunk(k, 2, w_scat=True, pf=True, w_gath=True)
        chunk(k + 1, 3, w_scat=True, pf=True, w_gath=True)
        chunk(k + 2, 0, w_scat=True, pf=True, w_gath=True)
        chunk(k + 3, 1, w_scat=True, pf=True, w_gath=True)
        return _

    chunk(0, 0, w_scat=False, pf=True, w_gath=False)
    chunk(1, 1, w_scat=False, pf=True, w_gath=True)
    lax.fori_loop(0, (RPW - 4) // 4, body, 0)
    chunk(RPW - 2, 2, w_scat=True, pf=True, w_gath=True)
    chunk(RPW - 1, 3, w_scat=True, pf=False, w_gath=True)
    pltpu.make_async_copy(xs_hbm.at[idx4.at[3]], rows2.at[1], sem_g1).wait()
    pltpu.async_copy(rows2.at[1], acc_sh.at[didx4.at[3]], sem_s1, add=True)
    pltpu.make_async_copy(rows2.at[0], acc_sh.at[didx4.at[2]], sem_s0).wait()
    pltpu.make_async_copy(rows2.at[1], acc_sh.at[didx4.at[3]], sem_s1).wait()
    plsc.subcore_barrier()
    pltpu.sync_copy(acc_sh.at[pl.ds(s * TROWS, TROWS)],
                    out_hbm.at[c, pl.ds(s * TROWS, TROWS)])


@functools.partial(
    pl.kernel,
    out_type=jax.ShapeDtypeStruct((L * B, EMB), jnp.float32),
    mesh=_mesh,
    scratch_types=[
        pltpu.VMEM((128,), jnp.int32),
        pltpu.VMEM((128, EMB), jnp.float32),
        pltpu.SemaphoreType.DMA,
    ],
)
def _seq_gather_sc(emb_hbm, seq_hbm, out_hbm, idx_v, rows, sem):
    c = lax.axis_index("c")
    s = lax.axis_index("s")
    wid = c * NS + s
    nb = jnp.where(wid < 16, 13, 12)
    base = jnp.where(wid < 16, wid * 13, 208 + (wid - 16) * 12)

    def body(k, _):
        r = base + k
        pltpu.sync_copy(seq_hbm.at[r], idx_v)
        pltpu.async_copy(emb_hbm.at[idx_v], rows, sem).wait()
        pltpu.sync_copy(rows, out_hbm.at[pl.ds(r * 128, 128)])
        return _

    lax.fori_loop(0, nb, body, 0)


def _prep_body(degp_ref, u_ref, i_ref, a_ref, b_ref, x0_ref, xs0_ref):
    deg = jnp.sum(degp_ref[...], axis=0)
    ab = lax.rsqrt(jnp.maximum(deg, 1.0))
    a_col = jnp.reshape(ab[0, :], (N_PAD, 1))
    b_col = jnp.reshape(ab[1, :], (N_PAD, 1))
    a_ref[...] = a_col
    b_ref[...] = b_col
    zeros_pad = jnp.zeros((N_PAD - N_NODES, EMB), dtype=jnp.float32)
    x0 = jnp.concatenate([u_ref[...], i_ref[...], zeros_pad], axis=0)
    x0_ref[...] = x0
    xs0_ref[...] = a_col * x0


def _prep_call(deg_partials, u_embs, i_embs):
    return pl.pallas_call(
        _prep_body,
        out_shape=(
            jax.ShapeDtypeStruct((N_PAD, 1), jnp.float32),
            jax.ShapeDtypeStruct((N_PAD, 1), jnp.float32),
            jax.ShapeDtypeStruct((N_PAD, EMB), jnp.float32),
            jax.ShapeDtypeStruct((N_PAD, EMB), jnp.float32),
        ),
    )(deg_partials, u_embs, i_embs)


def _combine_body(p_ref, a_ref, b_ref, acc_ref, xs_ref, accout_ref):
    x = b_ref[...] * jnp.sum(p_ref[...], axis=0)
    xs_ref[...] = a_ref[...] * x
    accout_ref[...] = acc_ref[...] + x


def _combine_call(partials, a, b, acc):
    return pl.pallas_call(
        _combine_body,
        out_shape=(
            jax.ShapeDtypeStruct((N_PAD, EMB), jnp.float32),
            jax.ShapeDtypeStruct((N_PAD, EMB), jnp.float32),
        ),
    )(partials, a, b, acc)


def _final_body(p_ref, b_ref, acc_ref, emb_ref):
    x2 = b_ref[...] * jnp.sum(p_ref[...], axis=0)
    emb_ref[...] = (acc_ref[...] + x2) * (1.0 / 3.0)


def _final_call(partials, b, acc):
    return pl.pallas_call(
        _final_body,
        out_shape=jax.ShapeDtypeStruct((N_PAD, EMB), jnp.float32),
    )(partials, b, acc)


def _gru_body(x_ref, wih0_ref, whh0_ref, bih0_ref, bhh0_ref,
              wih1_ref, bih1_ref, bhh1_ref, out_ref, h_ref):
    t = pl.program_id(0)

    @pl.when(t == 0)
    def _():
        h_ref[...] = jnp.zeros_like(h_ref)

    h = h_ref[...]
    gi = jnp.dot(x_ref[...], wih0_ref[...],
                 preferred_element_type=jnp.float32) + bih0_ref[...]
    gh = jnp.dot(h, whh0_ref[...],
                 preferred_element_type=jnp.float32) + bhh0_ref[...]
    r = jax.nn.sigmoid(gi[:, 0:EMB] + gh[:, 0:EMB])
    z = jax.nn.sigmoid(gi[:, EMB:2 * EMB] + gh[:, EMB:2 * EMB])
    n = jnp.tanh(gi[:, 2 * EMB:] + r * gh[:, 2 * EMB:])
    h_new = (1.0 - z) * n + z * h
    h_ref[...] = h_new

    @pl.when(t == pl.num_programs(0) - 1)
    def _():
        gi1 = jnp.dot(h_new, wih1_ref[...],
                      preferred_element_type=jnp.float32) + bih1_ref[...]
        r1 = jax.nn.sigmoid(gi1[:, 0:EMB] + bhh1_ref[:, 0:EMB])
        z1 = jax.nn.sigmoid(gi1[:, EMB:2 * EMB] + bhh1_ref[:, EMB:2 * EMB])
        n1 = jnp.tanh(gi1[:, 2 * EMB:] + r1 * bhh1_ref[:, 2 * EMB:])
        out_ref[...] = (1.0 - z1) * n1


def _gru_call(de_emb_flat, wih0t, whh0t, bih0, bhh0, wih1t, bih1, bhh1):
    full = lambda s: pl.BlockSpec(s, lambda t: (0,) * len(s))
    return pl.pallas_call(
        _gru_body,
        grid=(L,),
        in_specs=[
            pl.BlockSpec((B, EMB), lambda t: (t, 0)),
            full((EMB, H3)), full((EMB, H3)), full((1, H3)), full((1, H3)),
            full((EMB, H3)), full((1, H3)), full((1, H3)),
        ],
        out_specs=pl.BlockSpec((B, EMB), lambda t: (0, 0)),
        out_shape=jax.ShapeDtypeStruct((B, EMB), jnp.float32),
        scratch_shapes=[pltpu.VMEM((B, EMB), jnp.float32)],
    )(de_emb_flat, wih0t, whh0t, bih0, bhh0, wih1t, bih1, bhh1)


def kernel(u_embs, i_embs, edge_index, user_seq,
           W_ih0, W_hh0, b_ih0, b_hh0, W_ih1, W_hh1, b_ih1, b_hh1):
    del W_hh1
    edge32 = edge_index.astype(jnp.int32)
    pad = 10000 + (jnp.arange(E_PAD - E, dtype=jnp.int32) % (N_PAD - N_NODES))
    src2d = jnp.concatenate([edge32[0], pad]).reshape(EROWS, 128)
    dst2d = jnp.concatenate([edge32[1], pad]).reshape(EROWS, 128)
    zz = jnp.zeros((TROWS, EMB), jnp.float32)
    zhist = jnp.zeros((16 * HBINS + 16,), jnp.float32)

    deg_partials = _degrees_sc(src2d, dst2d, zhist)
    a, b, x0, xs0 = _prep_call(deg_partials, u_embs, i_embs)

    p1 = _smooth_sc(xs0, src2d, dst2d, zz)
    xs1, acc1 = _combine_call(p1, a, b, x0)
    p2 = _smooth_sc(xs1, src2d, dst2d, zz)
    emb_pad = _final_call(p2, b, acc1)

    seq2d = user_seq.astype(jnp.int32).T.reshape(SEQROWS, 128)
    de_emb = _seq_gather_sc(emb_pad, seq2d)

    h = _gru_call(
        de_emb,
        W_ih0.T, W_hh0.T, b_ih0.reshape(1, H3), b_hh0.reshape(1, H3),
        W_ih1.T, b_ih1.reshape(1, H3), b_hh1.reshape(1, H3),
    )
    return (h, emb_pad[:N_NODES])
